# baseline (speedup 1.0000x reference)
import functools

import jax
import jax.numpy as jnp
from jax.experimental import pallas as pl


NUM_CODES = 1024
CODE_DIM = 256
COMMITMENT_COST = 0.25
ROWS = 4096
SUBS = 8


def _vq_sub(z, z_sq, cb, c_sq, k_total):
    dot = jax.lax.dot_general(
        z, cb, (((1,), (1,)), ((), ())),
        preferred_element_type=jnp.float32)
    dist = z_sq - 2 * dot + c_sq
    mval = jnp.min(dist, axis=-1, keepdims=True)
    iota_k = jax.lax.broadcasted_iota(jnp.int32, dist.shape, 1)
    best_idx = jnp.min(jnp.where(dist == mval, iota_k, k_total),
                       axis=-1, keepdims=True)
    onehot = (iota_k == best_idx).astype(jnp.float32)
    z_q = jax.lax.dot_general(
        onehot, cb, (((1,), (0,)), ((), ())),
        preferred_element_type=jnp.float32)
    diff = z_q - z
    return best_idx[:, 0].astype(jnp.int32), z + diff, jnp.sum(diff * diff)


def _vq_body(z_ref, zsq_ref, cb_ref, csq_ref, zq_ref, idx_ref, loss_ref):
    cb = cb_ref[...]
    c_sq = csq_ref[...]
    k_total = cb.shape[0]
    sub_rows = z_ref.shape[0] // SUBS

    part = None
    for s in range(SUBS):
        sl = pl.ds(s * sub_rows, sub_rows)
        idx_s, zq_s, loss_s = _vq_sub(
            z_ref[sl, :], zsq_ref[sl, :], cb, c_sq, k_total)
        zq_ref[sl, :] = zq_s
        idx_ref[sl, :] = idx_s[:, None]
        part = loss_s if part is None else part + loss_s

    part = part.reshape(1, 1)

    @pl.when(pl.program_id(0) == 0)
    def _init():
        loss_ref[...] = part

    @pl.when(pl.program_id(0) != 0)
    def _acc():
        loss_ref[...] += part


@functools.partial(jax.jit, static_argnames=())
def kernel(z_e, codebook):
    B, N, D = z_e.shape
    K = codebook.shape[0]
    flat = z_e.reshape(B * N, D)
    nblk = (B * N) // ROWS
    z_sq = jnp.sum(z_e ** 2, axis=-1, keepdims=True).reshape(B * N, 1)
    c_sq = jnp.sum(codebook ** 2, axis=-1).reshape(1, K)

    zq_st, idx, loss_sum = pl.pallas_call(
        _vq_body,
        grid=(nblk,),
        in_specs=[
            pl.BlockSpec((ROWS, D), lambda i: (i, 0)),
            pl.BlockSpec((ROWS, 1), lambda i: (i, 0)),
            pl.BlockSpec((K, D), lambda i: (0, 0)),
            pl.BlockSpec((1, K), lambda i: (0, 0)),
        ],
        out_specs=[
            pl.BlockSpec((ROWS, D), lambda i: (i, 0)),
            pl.BlockSpec((ROWS, 1), lambda i: (i, 0)),
            pl.BlockSpec((1, 1), lambda i: (0, 0)),
        ],
        out_shape=[
            jax.ShapeDtypeStruct((B * N, D), jnp.float32),
            jax.ShapeDtypeStruct((B * N, 1), jnp.int32),
            jax.ShapeDtypeStruct((1, 1), jnp.float32),
        ],
    )(flat, z_sq, codebook, c_sq)

    mean_loss = loss_sum[0, 0] / (B * N * D)
    vq_loss = mean_loss + COMMITMENT_COST * mean_loss
    return (zq_st.reshape(B, N, D), idx.reshape(B, N), vq_loss)


# speedup vs baseline: 1.0032x; 1.0032x over previous
import functools

import jax
import jax.numpy as jnp
from jax.experimental import pallas as pl


NUM_CODES = 1024
CODE_DIM = 256
COMMITMENT_COST = 0.25
ROWS = 2048
SUBS = 8


def _vq_sub(z, z_sq, cb, c_sq, k_total):
    dot = jax.lax.dot_general(
        z, cb, (((1,), (1,)), ((), ())),
        preferred_element_type=jnp.float32)
    dist = z_sq - 2 * dot + c_sq
    mval = jnp.min(dist, axis=-1, keepdims=True)
    iota_k = jax.lax.broadcasted_iota(jnp.int32, dist.shape, 1)
    best_idx = jnp.min(jnp.where(dist == mval, iota_k, k_total),
                       axis=-1, keepdims=True)
    onehot = (iota_k == best_idx).astype(jnp.float32)
    z_q = jax.lax.dot_general(
        onehot, cb, (((1,), (0,)), ((), ())),
        preferred_element_type=jnp.float32)
    diff = z_q - z
    return best_idx[:, 0].astype(jnp.int32), z + diff, jnp.sum(diff * diff)


def _vq_body(z_ref, zsq_ref, cb_ref, csq_ref, zq_ref, idx_ref, loss_ref):
    cb = cb_ref[...]
    c_sq = csq_ref[...]
    k_total = cb.shape[0]
    sub_rows = z_ref.shape[0] // SUBS

    part = None
    for s in range(SUBS):
        sl = pl.ds(s * sub_rows, sub_rows)
        idx_s, zq_s, loss_s = _vq_sub(
            z_ref[sl, :], zsq_ref[sl, :], cb, c_sq, k_total)
        zq_ref[sl, :] = zq_s
        idx_ref[sl, :] = idx_s[:, None]
        part = loss_s if part is None else part + loss_s

    part = part.reshape(1, 1)

    @pl.when(pl.program_id(0) == 0)
    def _init():
        loss_ref[...] = part

    @pl.when(pl.program_id(0) != 0)
    def _acc():
        loss_ref[...] += part


@functools.partial(jax.jit, static_argnames=())
def kernel(z_e, codebook):
    B, N, D = z_e.shape
    K = codebook.shape[0]
    flat = z_e.reshape(B * N, D)
    nblk = (B * N) // ROWS
    z_sq = jnp.sum(z_e ** 2, axis=-1, keepdims=True).reshape(B * N, 1)
    c_sq = jnp.sum(codebook ** 2, axis=-1).reshape(1, K)

    zq_st, idx, loss_sum = pl.pallas_call(
        _vq_body,
        grid=(nblk,),
        in_specs=[
            pl.BlockSpec((ROWS, D), lambda i: (i, 0)),
            pl.BlockSpec((ROWS, 1), lambda i: (i, 0)),
            pl.BlockSpec((K, D), lambda i: (0, 0)),
            pl.BlockSpec((1, K), lambda i: (0, 0)),
        ],
        out_specs=[
            pl.BlockSpec((ROWS, D), lambda i: (i, 0)),
            pl.BlockSpec((ROWS, 1), lambda i: (i, 0)),
            pl.BlockSpec((1, 1), lambda i: (0, 0)),
        ],
        out_shape=[
            jax.ShapeDtypeStruct((B * N, D), jnp.float32),
            jax.ShapeDtypeStruct((B * N, 1), jnp.int32),
            jax.ShapeDtypeStruct((1, 1), jnp.float32),
        ],
    )(flat, z_sq, codebook, c_sq)

    mean_loss = loss_sum[0, 0] / (B * N * D)
    vq_loss = mean_loss + COMMITMENT_COST * mean_loss
    return (zq_st.reshape(B, N, D), idx.reshape(B, N), vq_loss)
